# baseline (speedup 1.0000x reference)
import jax
import jax.numpy as jnp
from jax import lax
from jax.experimental import pallas as pl
from jax.experimental.pallas import tpu as pltpu
from jax.experimental.pallas import tpu_sc as plsc

N = 10000
N_PAD = 10240
E = 320000
D = 128
ALPHA = 0.2

NC, NS = 2, 16
NW = NC * NS
E_W = E // NW
CHUNK = 80
NCHUNK = E_W // CHUNK
ROWS_T = N // NS


def _mm_body(x_ref, waa_ref, s1_ref, s2_ref):
    s = jnp.dot(x_ref[...], waa_ref[...], preferred_element_type=jnp.float32)
    s1_ref[...] = s[:, :1]
    s2_ref[...] = s[:, 1:]


def _tc_logits(x, W, aa):
    grid = 10
    blk = N // grid
    waa = None

    def _waa_body(w_ref, aa_ref, o_ref):
        o_ref[...] = jnp.dot(w_ref[...], aa_ref[...],
                             preferred_element_type=jnp.float32)

    waa = pl.pallas_call(
        _waa_body,
        in_specs=[pl.BlockSpec((D, D), lambda: (0, 0)),
                  pl.BlockSpec((D, 2), lambda: (0, 0))],
        out_specs=pl.BlockSpec((D, 2), lambda: (0, 0)),
        out_shape=jax.ShapeDtypeStruct((D, 2), jnp.float32),
    )(W, aa)
    return pl.pallas_call(
        _mm_body,
        grid=(grid,),
        in_specs=[
            pl.BlockSpec((blk, D), lambda i: (i, 0)),
            pl.BlockSpec((D, 2), lambda i: (0, 0)),
        ],
        out_specs=[pl.BlockSpec((blk, 1), lambda i: (i, 0)),
                   pl.BlockSpec((blk, 1), lambda i: (i, 0))],
        out_shape=[jax.ShapeDtypeStruct((N, 1), jnp.float32),
                   jax.ShapeDtypeStruct((N, 1), jnp.float32)],
    )(x, waa)


def _fin_body(p_ref, d_ref, w_ref, o_ref):
    p = p_ref[...]
    acc = p[0] + p[1]
    y = jnp.dot(acc, w_ref[...], preferred_element_type=jnp.float32)
    den = jnp.sum(d_ref[...], axis=0)
    o = y / jnp.maximum(den, 1e-38)[:, None]
    o_ref[...] = jnp.where(o > 0, o, jnp.exp(jnp.minimum(o, 0.0)) - 1.0)


def _tc_finish(out_part, denom_part, W):
    grid = 10
    blk = N_PAD // grid
    return pl.pallas_call(
        _fin_body,
        grid=(grid,),
        in_specs=[
            pl.BlockSpec((2, blk, D), lambda i: (0, i, 0)),
            pl.BlockSpec((NC, blk), lambda i: (0, i)),
            pl.BlockSpec((D, D), lambda i: (0, 0)),
        ],
        out_specs=pl.BlockSpec((blk, D), lambda i: (i, 0)),
        out_shape=jax.ShapeDtypeStruct((N, D), jnp.float32),
    )(out_part, denom_part, W)


def _sc_body(x_hbm, s1_hbm, s2_hbm, edge_hbm,
             out_hbm, den_hbm,
             idx_v, s1c_v, s2c_v, exc_v, rows_v, out_sh, den_sh,
             isem, s1sem, s2sem, rsem, wsem, dsem):
    cid = lax.axis_index("c")
    sid = lax.axis_index("s")
    g = cid * NS + sid

    zeros16 = jnp.zeros((16,), jnp.float32)

    def _zero_rows(i, _):
        for f in range(8):
            rows_v[0, i, pl.ds(f * 16, 16)] = zeros16
        return _
    lax.fori_loop(0, CHUNK, _zero_rows, None)
    for k in range(CHUNK // 16):
        exc_v[0, pl.ds(k * 16, 16)] = zeros16

    @pl.when(sid < (N_PAD - N) // CHUNK)
    def _():
        pltpu.sync_copy(rows_v.at[0],
                        out_hbm.at[cid].at[pl.ds(N + sid * CHUNK, CHUNK)])

    for k in range(7):
        pltpu.sync_copy(rows_v.at[0],
                        out_sh.at[pl.ds(sid * 624 + k * CHUNK, CHUNK)])
    pltpu.sync_copy(rows_v.at[0, pl.ds(0, 64)],
                    out_sh.at[pl.ds(sid * 624 + 7 * CHUNK, 64)])
    @pl.when(sid == 0)
    def _():
        pltpu.sync_copy(rows_v.at[0, pl.ds(0, 16)],
                        out_sh.at[pl.ds(16 * 624, 16)])
    for k in range(8):
        pltpu.sync_copy(exc_v.at[0],
                        den_sh.at[pl.ds(sid * 640 + k * CHUNK, CHUNK)])
    plsc.subcore_barrier()

    c0 = g * NCHUNK

    def _idx_copy(c, j):
        pltpu.async_copy(edge_hbm.at[0].at[c0 + c], idx_v.at[j, 0],
                         isem.at[j])
        pltpu.async_copy(edge_hbm.at[1].at[c0 + c], idx_v.at[j, 1],
                         isem.at[j])

    def _idx_wait(c, j):
        pltpu.make_async_copy(edge_hbm.at[0].at[c0 + c], idx_v.at[j, 0],
                              isem.at[j]).wait()
        pltpu.make_async_copy(edge_hbm.at[1].at[c0 + c], idx_v.at[j, 1],
                              isem.at[j]).wait()

    for t in range(6):
        _idx_copy(t, t)

    def _issue(c, b, q, p):
        pltpu.async_copy(s1_hbm.at[idx_v.at[p, 0]], s1c_v.at[b], s1sem.at[b])
        pltpu.async_copy(s2_hbm.at[idx_v.at[p, 1]], s2c_v.at[b], s2sem.at[b])
        pltpu.async_copy(x_hbm.at[idx_v.at[p, 1]], rows_v.at[q], rsem.at[q])

    for t in range(2):
        _idx_wait(t, t)
        _issue(t, t, t, t)

    def _do_chunk(c, b, q, p, tail):
        p2 = (p + 2) % 8
        p6 = (p + 6) % 8
        q2 = (q + 2) % 4
        pltpu.make_async_copy(s1_hbm.at[idx_v.at[p, 0]], s1c_v.at[b],
                              s1sem.at[b]).wait()
        pltpu.make_async_copy(s2_hbm.at[idx_v.at[p, 1]], s2c_v.at[b],
                              s2sem.at[b]).wait()
        pltpu.make_async_copy(x_hbm.at[idx_v.at[p, 1]], rows_v.at[q],
                              rsem.at[q]).wait()

        def _wait_den():
            pltpu.make_async_copy(exc_v.at[b], den_sh.at[idx_v.at[p, 0]],
                                  dsem.at[b]).wait()
        if tail:
            if c >= 2:
                _wait_den()
        else:
            @pl.when(c >= 2)
            def _():
                _wait_den()

        for k in range(CHUNK // 16):
            sl = pl.ds(k * 16, 16)
            e = s1c_v[b, sl] + s2c_v[b, sl]
            e = jnp.where(e > 0, e, ALPHA * e)
            exc_v[b, sl] = jnp.exp(e)

        def _issue_s(cn):
            pltpu.make_async_copy(edge_hbm.at[0].at[c0 + cn], idx_v.at[p2, 0],
                                  isem.at[p2]).wait()
            pltpu.make_async_copy(edge_hbm.at[1].at[c0 + cn], idx_v.at[p2, 1],
                                  isem.at[p2]).wait()
            pltpu.async_copy(s1_hbm.at[idx_v.at[p2, 0]], s1c_v.at[b],
                             s1sem.at[b])
            pltpu.async_copy(s2_hbm.at[idx_v.at[p2, 1]], s2c_v.at[b],
                             s2sem.at[b])
        if tail:
            if c + 2 < NCHUNK:
                _issue_s(c + 2)
        else:
            @pl.when(c + 2 < NCHUNK)
            def _():
                _issue_s(c + 2)

        def _grp(k, _):
            exg = exc_v[b, pl.ds(k * 16, 16)]
            for r in range(16):
                exs = exg[r]
                row = k * 16 + r
                for f in range(8):
                    sl = pl.ds(f * 16, 16)
                    rows_v[q, row, sl] = rows_v[q, row, sl] * exs
            return _
        lax.fori_loop(0, CHUNK // 16, _grp, None)


        pltpu.async_copy(exc_v.at[b], den_sh.at[idx_v.at[p, 0]], dsem.at[b],
                         add=True)
        pltpu.async_copy(rows_v.at[q], out_sh.at[idx_v.at[p, 0]], wsem.at[q],
                         add=True)

        def _wait_prev():
            pltpu.make_async_copy(rows_v.at[q2], out_sh.at[idx_v.at[p, 0]],
                                  wsem.at[q2]).wait()

        def _issue_rows():
            pltpu.async_copy(x_hbm.at[idx_v.at[p2, 1]], rows_v.at[q2],
                             rsem.at[q2])
        if tail:
            if c >= 2:
                _wait_prev()
            if c + 2 < NCHUNK:
                _issue_rows()
            if c + 6 < NCHUNK:
                _idx_copy(c + 6, p6)
        else:
            @pl.when(c >= 2)
            def _():
                _wait_prev()

            @pl.when(c + 2 < NCHUNK)
            def _():
                _issue_rows()

            @pl.when(c + 6 < NCHUNK)
            def _():
                _idx_copy(c + 6, p6)

    def _eight(t, _):
        for u in range(8):
            _do_chunk(8 * t + u, u % 2, u % 4, u, False)
        return _
    lax.fori_loop(0, NCHUNK // 8, _eight, None)
    for c in range(8 * (NCHUNK // 8), NCHUNK):
        _do_chunk(c, c % 2, c % 4, c % 8, True)

    for ql in ((NCHUNK - 2) % 4, (NCHUNK - 1) % 4):
        pltpu.make_async_copy(rows_v.at[ql], out_sh.at[idx_v.at[0, 0]],
                              wsem.at[ql]).wait()
    for b in ((NCHUNK - 2) % 2, (NCHUNK - 1) % 2):
        pltpu.make_async_copy(exc_v.at[b], den_sh.at[idx_v.at[0, 0]],
                              dsem.at[b]).wait()
    plsc.subcore_barrier()
    pltpu.sync_copy(out_sh.at[pl.ds(sid * 624, 624)],
                    out_hbm.at[cid].at[pl.ds(sid * 624, 624)])
    pltpu.sync_copy(den_sh.at[pl.ds(sid * 640, 640)],
                    den_hbm.at[cid].at[pl.ds(sid * 640, 640)])
    @pl.when(sid == 0)
    def _():
        pltpu.sync_copy(out_sh.at[pl.ds(16 * 624, 16)],
                        out_hbm.at[cid].at[pl.ds(16 * 624, 16)])


def _sc_edge(x, s1, s2, edge3):
    mesh = plsc.VectorSubcoreMesh(core_axis_name="c", subcore_axis_name="s",
                                  num_cores=NC, num_subcores=NS)
    f = pl.kernel(
        _sc_body,
        out_type=[
            jax.ShapeDtypeStruct((NC, N_PAD, D), jnp.float32),
            jax.ShapeDtypeStruct((NC, N_PAD), jnp.float32),
        ],
        mesh=mesh,
        compiler_params=pltpu.CompilerParams(needs_layout_passes=False),
        scratch_types=[
            pltpu.VMEM((8, 2, CHUNK), jnp.int32),
            pltpu.VMEM((2, CHUNK), jnp.float32),
            pltpu.VMEM((2, CHUNK), jnp.float32),
            pltpu.VMEM((2, CHUNK), jnp.float32),
            pltpu.VMEM((4, CHUNK, D), jnp.float32),
            pltpu.VMEM_SHARED((N, D), jnp.float32),
            pltpu.VMEM_SHARED((N_PAD,), jnp.float32),
            pltpu.SemaphoreType.DMA((8,)),
            pltpu.SemaphoreType.DMA((2,)),
            pltpu.SemaphoreType.DMA((2,)),
            pltpu.SemaphoreType.DMA((4,)),
            pltpu.SemaphoreType.DMA((4,)),
            pltpu.SemaphoreType.DMA((2,)),
        ],
    )
    return f(x, s1, s2, edge3)


def kernel(input, edge, W, a):
    x = input.astype(jnp.float32)
    aa = jnp.concatenate([a[:D], a[D:]], axis=1)
    s1, s2 = _tc_logits(x, W, aa)
    edge2 = edge.astype(jnp.int32).reshape(2, E // CHUNK, CHUNK)
    out_part, denom_part = _sc_edge(x, s1.reshape(N), s2.reshape(N), edge2)
    return _tc_finish(out_part, denom_part, W)


# speedup vs baseline: 1.0883x; 1.0883x over previous
import jax
import jax.numpy as jnp
from jax import lax
from jax.experimental import pallas as pl
from jax.experimental.pallas import tpu as pltpu
from jax.experimental.pallas import tpu_sc as plsc

N = 10000
N_PAD = 10240
E = 320000
D = 128
ALPHA = 0.2

NC, NS = 2, 16
NW = NC * NS
E_W = E // NW
CHUNK = 80
NCHUNK = E_W // CHUNK
ROWS_T = N // NS


def _mm_body(x_ref, waa_ref, s_ref):
    s_ref[...] = jnp.dot(x_ref[...], waa_ref[...],
                         preferred_element_type=jnp.float32)


def _tc_logits(x, W, aa):
    grid = 10
    blk = N // grid
    waa = None

    def _waa_body(w_ref, aa_ref, o_ref):
        o_ref[...] = jnp.dot(w_ref[...], aa_ref[...],
                             preferred_element_type=jnp.float32)

    waa = pl.pallas_call(
        _waa_body,
        in_specs=[pl.BlockSpec((D, D), lambda: (0, 0)),
                  pl.BlockSpec((D, 2), lambda: (0, 0))],
        out_specs=pl.BlockSpec((D, 2), lambda: (0, 0)),
        out_shape=jax.ShapeDtypeStruct((D, 2), jnp.float32),
    )(W, aa)
    return pl.pallas_call(
        _mm_body,
        grid=(grid,),
        in_specs=[
            pl.BlockSpec((blk, D), lambda i: (i, 0)),
            pl.BlockSpec((D, 2), lambda i: (0, 0)),
        ],
        out_specs=pl.BlockSpec((blk, 2), lambda i: (i, 0)),
        out_shape=jax.ShapeDtypeStruct((N, 2), jnp.float32),
    )(x, waa)


def _fin_body(p_ref, d_ref, w_ref, o_ref):
    p = p_ref[...]
    acc = p[0] + p[1]
    y = jnp.dot(acc, w_ref[...], preferred_element_type=jnp.float32)
    den = jnp.sum(d_ref[...], axis=0)
    o = y / jnp.maximum(den, 1e-38)[:, None]
    o_ref[...] = jnp.where(o > 0, o, jnp.exp(jnp.minimum(o, 0.0)) - 1.0)


def _tc_finish(out_part, denom_part, W):
    grid = 10
    blk = N_PAD // grid
    return pl.pallas_call(
        _fin_body,
        grid=(grid,),
        in_specs=[
            pl.BlockSpec((2, blk, D), lambda i: (0, i, 0)),
            pl.BlockSpec((NC, blk), lambda i: (0, i)),
            pl.BlockSpec((D, D), lambda i: (0, 0)),
        ],
        out_specs=pl.BlockSpec((blk, D), lambda i: (i, 0)),
        out_shape=jax.ShapeDtypeStruct((N, D), jnp.float32),
    )(out_part, denom_part, W)


def _sc_body(x_hbm, s1_hbm, s2_hbm, edge_hbm,
             out_hbm, den_hbm,
             idx_v, s1c_v, s2c_v, exc_v, rows_v, out_sh, den_sh,
             isem, s1sem, s2sem, rsem, wsem, dsem):
    cid = lax.axis_index("c")
    sid = lax.axis_index("s")
    g = cid * NS + sid

    zeros16 = jnp.zeros((16,), jnp.float32)

    def _zero_rows(i, _):
        for f in range(8):
            rows_v[0, i, pl.ds(f * 16, 16)] = zeros16
        return _
    lax.fori_loop(0, CHUNK, _zero_rows, None)
    for k in range(CHUNK // 16):
        exc_v[0, pl.ds(k * 16, 16)] = zeros16

    @pl.when(sid < (N_PAD - N) // CHUNK)
    def _():
        pltpu.sync_copy(rows_v.at[0],
                        out_hbm.at[cid].at[pl.ds(N + sid * CHUNK, CHUNK)])

    for k in range(7):
        pltpu.sync_copy(rows_v.at[0],
                        out_sh.at[pl.ds(sid * 624 + k * CHUNK, CHUNK)])
    pltpu.sync_copy(rows_v.at[0, pl.ds(0, 64)],
                    out_sh.at[pl.ds(sid * 624 + 7 * CHUNK, 64)])
    @pl.when(sid == 0)
    def _():
        pltpu.sync_copy(rows_v.at[0, pl.ds(0, 16)],
                        out_sh.at[pl.ds(16 * 624, 16)])
    for k in range(8):
        pltpu.sync_copy(exc_v.at[0],
                        den_sh.at[pl.ds(sid * 640 + k * CHUNK, CHUNK)])
    plsc.subcore_barrier()

    c0 = g * NCHUNK

    def _idx_copy(c, j):
        pltpu.async_copy(edge_hbm.at[0].at[c0 + c], idx_v.at[j, 0],
                         isem.at[j])
        pltpu.async_copy(edge_hbm.at[1].at[c0 + c], idx_v.at[j, 1],
                         isem.at[j])

    def _idx_wait(c, j):
        pltpu.make_async_copy(edge_hbm.at[0].at[c0 + c], idx_v.at[j, 0],
                              isem.at[j]).wait()
        pltpu.make_async_copy(edge_hbm.at[1].at[c0 + c], idx_v.at[j, 1],
                              isem.at[j]).wait()

    for t in range(6):
        _idx_copy(t, t)

    def _issue(c, b, q, p):
        pltpu.async_copy(s1_hbm.at[idx_v.at[p, 0]], s1c_v.at[b], s1sem.at[b])
        pltpu.async_copy(s2_hbm.at[idx_v.at[p, 1]], s2c_v.at[b], s2sem.at[b])
        pltpu.async_copy(x_hbm.at[idx_v.at[p, 1]], rows_v.at[q], rsem.at[q])

    for t in range(2):
        _idx_wait(t, t)
        _issue(t, t, t, t)

    def _do_chunk(c, b, q, p, tail):
        p2 = (p + 2) % 8
        p6 = (p + 6) % 8
        q2 = (q + 2) % 4
        pltpu.make_async_copy(s1_hbm.at[idx_v.at[p, 0]], s1c_v.at[b],
                              s1sem.at[b]).wait()
        pltpu.make_async_copy(s2_hbm.at[idx_v.at[p, 1]], s2c_v.at[b],
                              s2sem.at[b]).wait()
        pltpu.make_async_copy(x_hbm.at[idx_v.at[p, 1]], rows_v.at[q],
                              rsem.at[q]).wait()

        def _wait_den():
            pltpu.make_async_copy(exc_v.at[b], den_sh.at[idx_v.at[p, 0]],
                                  dsem.at[b]).wait()
        if tail:
            if c >= 2:
                _wait_den()
        else:
            @pl.when(c >= 2)
            def _():
                _wait_den()

        for k in range(CHUNK // 16):
            sl = pl.ds(k * 16, 16)
            e = s1c_v[b, sl] + s2c_v[b, sl]
            e = jnp.where(e > 0, e, ALPHA * e)
            exc_v[b, sl] = jnp.exp(e)

        def _issue_s(cn):
            pltpu.make_async_copy(edge_hbm.at[0].at[c0 + cn], idx_v.at[p2, 0],
                                  isem.at[p2]).wait()
            pltpu.make_async_copy(edge_hbm.at[1].at[c0 + cn], idx_v.at[p2, 1],
                                  isem.at[p2]).wait()
            pltpu.async_copy(s1_hbm.at[idx_v.at[p2, 0]], s1c_v.at[b],
                             s1sem.at[b])
            pltpu.async_copy(s2_hbm.at[idx_v.at[p2, 1]], s2c_v.at[b],
                             s2sem.at[b])
        if tail:
            if c + 2 < NCHUNK:
                _issue_s(c + 2)
        else:
            @pl.when(c + 2 < NCHUNK)
            def _():
                _issue_s(c + 2)

        def _wait_prev():
            pltpu.make_async_copy(rows_v.at[q2], out_sh.at[idx_v.at[p, 0]],
                                  wsem.at[q2]).wait()

        def _issue_rows():
            pltpu.async_copy(x_hbm.at[idx_v.at[p2, 1]], rows_v.at[q2],
                             rsem.at[q2])
        if tail:
            if c >= 2:
                _wait_prev()
            if c + 2 < NCHUNK:
                _issue_rows()
            if c + 6 < NCHUNK:
                _idx_copy(c + 6, p6)
        else:
            @pl.when(c >= 2)
            def _():
                _wait_prev()

            @pl.when(c + 2 < NCHUNK)
            def _():
                _issue_rows()

            @pl.when(c + 6 < NCHUNK)
            def _():
                _idx_copy(c + 6, p6)

        def _grp(k, _):
            exg = exc_v[b, pl.ds(k * 16, 16)]
            for r in range(16):
                exs = exg[r]
                row = k * 16 + r
                for f in range(8):
                    sl = pl.ds(f * 16, 16)
                    rows_v[q, row, sl] = rows_v[q, row, sl] * exs
            return _
        lax.fori_loop(0, CHUNK // 16, _grp, None)

        pltpu.async_copy(exc_v.at[b], den_sh.at[idx_v.at[p, 0]], dsem.at[b],
                         add=True)
        pltpu.async_copy(rows_v.at[q], out_sh.at[idx_v.at[p, 0]], wsem.at[q],
                         add=True)

    def _eight(t, _):
        for u in range(8):
            _do_chunk(8 * t + u, u % 2, u % 4, u, False)
        return _
    lax.fori_loop(0, NCHUNK // 8, _eight, None)
    for c in range(8 * (NCHUNK // 8), NCHUNK):
        _do_chunk(c, c % 2, c % 4, c % 8, True)

    for ql in ((NCHUNK - 2) % 4, (NCHUNK - 1) % 4):
        pltpu.make_async_copy(rows_v.at[ql], out_sh.at[idx_v.at[0, 0]],
                              wsem.at[ql]).wait()
    for b in ((NCHUNK - 2) % 2, (NCHUNK - 1) % 2):
        pltpu.make_async_copy(exc_v.at[b], den_sh.at[idx_v.at[0, 0]],
                              dsem.at[b]).wait()
    plsc.subcore_barrier()
    pltpu.sync_copy(out_sh.at[pl.ds(sid * 624, 624)],
                    out_hbm.at[cid].at[pl.ds(sid * 624, 624)])
    pltpu.sync_copy(den_sh.at[pl.ds(sid * 640, 640)],
                    den_hbm.at[cid].at[pl.ds(sid * 640, 640)])
    @pl.when(sid == 0)
    def _():
        pltpu.sync_copy(out_sh.at[pl.ds(16 * 624, 16)],
                        out_hbm.at[cid].at[pl.ds(16 * 624, 16)])


def _sc_edge(x, s1, s2, edge3):
    mesh = plsc.VectorSubcoreMesh(core_axis_name="c", subcore_axis_name="s",
                                  num_cores=NC, num_subcores=NS)
    f = pl.kernel(
        _sc_body,
        out_type=[
            jax.ShapeDtypeStruct((NC, N_PAD, D), jnp.float32),
            jax.ShapeDtypeStruct((NC, N_PAD), jnp.float32),
        ],
        mesh=mesh,
        compiler_params=pltpu.CompilerParams(needs_layout_passes=False),
        scratch_types=[
            pltpu.VMEM((8, 2, CHUNK), jnp.int32),
            pltpu.VMEM((2, CHUNK), jnp.float32),
            pltpu.VMEM((2, CHUNK), jnp.float32),
            pltpu.VMEM((2, CHUNK), jnp.float32),
            pltpu.VMEM((4, CHUNK, D), jnp.float32),
            pltpu.VMEM_SHARED((N, D), jnp.float32),
            pltpu.VMEM_SHARED((N_PAD,), jnp.float32),
            pltpu.SemaphoreType.DMA((8,)),
            pltpu.SemaphoreType.DMA((2,)),
            pltpu.SemaphoreType.DMA((2,)),
            pltpu.SemaphoreType.DMA((4,)),
            pltpu.SemaphoreType.DMA((4,)),
            pltpu.SemaphoreType.DMA((2,)),
        ],
    )
    return f(x, s1, s2, edge3)


def kernel(input, edge, W, a):
    x = input.astype(jnp.float32)
    aa = jnp.concatenate([a[:D], a[D:]], axis=1)
    s = _tc_logits(x, W, aa)
    edge2 = edge.astype(jnp.int32).reshape(2, E // CHUNK, CHUNK)
    out_part, denom_part = _sc_edge(x, s[:, 0], s[:, 1], edge2)
    return _tc_finish(out_part, denom_part, W)
